# baseline (device time: 30846 ns/iter reference)
import jax
import jax.numpy as jnp
from jax import lax
from jax.experimental import pallas as pl
from jax.experimental.pallas import tpu as pltpu

N_DEV = 16
HALVES = 2


def kernel(x):
    m, n = x.shape
    chunk = m // N_DEV
    sub = chunk // HALVES
    n_sub = N_DEV * HALVES

    def body(x_ref, out_ref, stage_ref, rs_ref, ag_ref,
             rs_send_sems, rs_recv_sems, ag_send_sems, ag_recv_sems):
        my = lax.axis_index("i")

        stage_ref[...] = x_ref[...].reshape(n_sub, sub, n).astype(jnp.bfloat16)

        barrier_sem = pltpu.get_barrier_semaphore()
        for o in range(1, N_DEV):
            pl.semaphore_signal(
                barrier_sem, inc=1,
                device_id=((my + o) % N_DEV,),
                device_id_type=pl.DeviceIdType.MESH,
            )
        pl.semaphore_wait(barrier_sem, N_DEV - 1)

        sends = []
        for h in range(HALVES):
            for o in range(1, N_DEV):
                dest = (my + o) % N_DEV
                rdma = pltpu.make_async_remote_copy(
                    src_ref=stage_ref.at[HALVES * dest + h],
                    dst_ref=rs_ref.at[HALVES * my + h],
                    send_sem=rs_send_sems.at[HALVES * o + h],
                    recv_sem=rs_recv_sems.at[HALVES * my + h],
                    device_id=(dest,),
                    device_id_type=pl.DeviceIdType.MESH,
                )
                rdma.start()
                sends.append(rdma)

        for h in range(HALVES):
            acc = stage_ref[HALVES * my + h].astype(jnp.float32)
            for o in range(1, N_DEV):
                src = (my + o) % N_DEV
                pltpu.make_async_remote_copy(
                    src_ref=stage_ref.at[0],
                    dst_ref=rs_ref.at[HALVES * src + h],
                    send_sem=rs_send_sems.at[0],
                    recv_sem=rs_recv_sems.at[HALVES * src + h],
                    device_id=(my,),
                    device_id_type=pl.DeviceIdType.MESH,
                ).wait_recv()
                acc = acc + rs_ref[HALVES * src + h].astype(jnp.float32)

            ag_ref[pl.ds(HALVES * my + h, 1)] = acc[None].astype(jnp.bfloat16)
            for o in range(1, N_DEV):
                dest = (my + o) % N_DEV
                rdma = pltpu.make_async_remote_copy(
                    src_ref=ag_ref.at[HALVES * my + h],
                    dst_ref=ag_ref.at[HALVES * my + h],
                    send_sem=ag_send_sems.at[HALVES * o + h],
                    recv_sem=ag_recv_sems.at[HALVES * my + h],
                    device_id=(dest,),
                    device_id_type=pl.DeviceIdType.MESH,
                )
                rdma.start()
                sends.append(rdma)
            out_ref[pl.ds(my * chunk + h * sub, sub)] = acc

        for h in range(HALVES):
            for o in range(1, N_DEV):
                src = (my + o) % N_DEV
                pltpu.make_async_remote_copy(
                    src_ref=ag_ref.at[0],
                    dst_ref=ag_ref.at[HALVES * src + h],
                    send_sem=ag_send_sems.at[0],
                    recv_sem=ag_recv_sems.at[HALVES * src + h],
                    device_id=(my,),
                    device_id_type=pl.DeviceIdType.MESH,
                ).wait_recv()
                out_ref[pl.ds(src * chunk + h * sub, sub)] = (
                    ag_ref[HALVES * src + h].astype(jnp.float32)
                )

        for rdma in sends:
            rdma.wait_send()

    return pl.pallas_call(
        body,
        out_shape=jax.ShapeDtypeStruct((m, n), jnp.float32),
        in_specs=[pl.BlockSpec(memory_space=pltpu.VMEM)],
        out_specs=pl.BlockSpec(memory_space=pltpu.VMEM),
        scratch_shapes=[
            pltpu.VMEM((n_sub, sub, n), jnp.bfloat16),
            pltpu.VMEM((n_sub, sub, n), jnp.bfloat16),
            pltpu.VMEM((n_sub, sub, n), jnp.bfloat16),
            pltpu.SemaphoreType.DMA((n_sub,)),
            pltpu.SemaphoreType.DMA((n_sub,)),
            pltpu.SemaphoreType.DMA((n_sub,)),
            pltpu.SemaphoreType.DMA((n_sub,)),
        ],
        compiler_params=pltpu.CompilerParams(collective_id=0),
    )(x)
